# baseline (speedup 1.0000x reference)
import functools

import jax
import jax.numpy as jnp
from jax import lax
from jax.experimental import pallas as pl
from jax.experimental.pallas import tpu as pltpu
from jax.experimental.pallas import tpu_sc as plsc

NUM_USERS = 35000
NUM_ITEMS = 15000
NUM_NODES = NUM_USERS + NUM_ITEMS
D = 64
HD = 32
K = 10
N_LAYERS = 3

NC = 2
NS = 16
NW = NC * NS

I_PAD = 16384
G_PAD = I_PAD * K
G_CHUNK = 512
G_SUB = G_CHUNK // 128
G_CHUNKS_PER_W = G_PAD // (NW * G_CHUNK)

E_PAD = 802816
E_CHUNK = 128
NBUF = 4
SB_CHUNKS = 28
E_SB_PER_TILE = 14
SB_EDGES = SB_CHUNKS * E_CHUNK
ROWS_PER_TILE = NUM_NODES // NS
ZCHUNK = 625

_mesh = plsc.VectorSubcoreMesh(
    core_axis_name="c", subcore_axis_name="s", num_cores=NC, num_subcores=NS
)
_sc_params = pltpu.CompilerParams(
    use_tc_tiling_on_sc=False, needs_layout_passes=False
)


@functools.partial(
    pl.kernel,
    mesh=_mesh,
    out_type=jax.ShapeDtypeStruct((G_PAD, D), jnp.float32),
    scratch_types=[
        pltpu.VMEM((G_CHUNKS_PER_W * G_SUB, 128), jnp.int32),
        [pltpu.VMEM((G_CHUNK, D), jnp.float32) for _ in range(2)],
        [pltpu.SemaphoreType.DMA for _ in range(2)],
        [pltpu.SemaphoreType.DMA for _ in range(2)],
    ],
    compiler_params=_sc_params,
)
def _sc_gather_entities(tbl_hbm, idx_hbm, out_hbm, idx_v, rows, sem_g, sem_w):
    c = lax.axis_index("c")
    s = lax.axis_index("s")
    wid = s * NC + c

    pltpu.sync_copy(
        idx_hbm.at[pl.ds(wid * G_CHUNKS_PER_W * G_SUB, G_CHUNKS_PER_W * G_SUB)],
        idx_v)

    def fire_gathers(j, b):
        return [pltpu.async_copy(tbl_hbm.at[idx_v.at[j * G_SUB + jj]],
                                 rows[b].at[pl.ds(jj * 128, 128)],
                                 sem_g[b])
                for jj in range(G_SUB)]

    gd = [fire_gathers(0, 0), fire_gathers(1, 1)]
    wb = [None, None]
    for j in range(G_CHUNKS_PER_W):
        b = j % 2
        for d_ in gd[b]:
            d_.wait()
        base = (wid * G_CHUNKS_PER_W + j) * G_CHUNK
        wb[b] = pltpu.async_copy(rows[b], out_hbm.at[pl.ds(base, G_CHUNK)],
                                 sem_w[b])
        if j + 2 < G_CHUNKS_PER_W:
            wb[b].wait()
            gd[b] = fire_gathers(j + 2, b)
    wb[0].wait()
    wb[1].wait()


_BI = 600


def _kg_body(ent_ref, item_ref, rel_ref, w_ref, b_ref, re_ref, o_ref):
    hi = jax.lax.Precision.HIGHEST
    item = item_ref[...]
    W1 = w_ref[0:D, :]
    W2 = w_ref[D : 2 * D, :]
    b = b_ref[...]
    re_tab = re_ref[...]
    itemW1 = lax.dot(item, W1, precision=hi)
    ent3 = ent_ref[...]
    kgr = rel_ref[...]

    es = []
    for k in range(K):
        ent_k = ent3[:, k, :]
        entW2 = lax.dot(ent_k, W2, precision=hi)
        iota24 = lax.broadcasted_iota(jnp.int32, (_BI, 24), 1)
        oh = (kgr[:, k : k + 1] == iota24).astype(jnp.float32)
        rel_k = lax.dot(oh, re_tab, precision=hi)
        e_k = jnp.sum((itemW1 + entW2 + b) * rel_k, axis=-1, keepdims=True)
        es.append(e_k)
    e = jnp.concatenate(es, axis=1)
    e = jnp.where(e > 0, e, 0.2 * e)
    m = jnp.max(e, axis=1, keepdims=True)
    p = jnp.exp(e - m)
    att = p / jnp.sum(p, axis=1, keepdims=True)

    out = item
    for k in range(K):
        out = out + att[:, k : k + 1] * ent3[:, k, :]
    o_ref[...] = out


def _kg_attention(ent3r, item_emb, kg_relations, fc_W, fc_b2, rel_pad):
    grid = (NUM_ITEMS // _BI,)
    return pl.pallas_call(
        _kg_body,
        grid=grid,
        in_specs=[
            pl.BlockSpec((_BI, K, D), lambda i: (i, 0, 0)),
            pl.BlockSpec((_BI, D), lambda i: (i, 0)),
            pl.BlockSpec((_BI, K), lambda i: (i, 0)),
            pl.BlockSpec((2 * D, D), lambda i: (0, 0)),
            pl.BlockSpec((1, D), lambda i: (0, 0)),
            pl.BlockSpec((24, D), lambda i: (0, 0)),
        ],
        out_specs=pl.BlockSpec((_BI, D), lambda i: (i, 0)),
        out_shape=jax.ShapeDtypeStruct((NUM_ITEMS, D), jnp.float32),
    )(ent3r, item_emb, kg_relations, fc_W, fc_b2, rel_pad)


@functools.partial(
    pl.kernel,
    mesh=_mesh,
    out_type=jax.ShapeDtypeStruct((N_LAYERS, NC, NUM_NODES, HD), jnp.float32),
    scratch_types=[
        pltpu.VMEM((SB_CHUNKS, 128), jnp.int32),
        pltpu.VMEM((SB_CHUNKS, 128), jnp.int32),
        pltpu.VMEM((SB_EDGES,), jnp.float32),
        [pltpu.VMEM((E_CHUNK, HD), jnp.float32) for _ in range(NBUF)],
        pltpu.VMEM_SHARED((NUM_NODES, HD), jnp.float32),
        [pltpu.SemaphoreType.DMA for _ in range(NBUF)],
        [pltpu.SemaphoreType.DMA for _ in range(NBUF)],
    ],
    compiler_params=_sc_params,
)
def _sc_propagate(t0_hbm, src_hbm, dst_hbm, w_hbm, z_hbm, out_hbm,
                  src_v, dst_v, w_v, rows, acc, sem_g, sem_s):
    c = lax.axis_index("c")
    s = lax.axis_index("s")

    @pl.loop(0, ROWS_PER_TILE // ZCHUNK)
    def _(z):
        pltpu.sync_copy(z_hbm, acc.at[pl.ds(s * ROWS_PER_TILE + z * ZCHUNK, ZCHUNK)])

    plsc.subcore_barrier()

    for l in range(N_LAYERS):
        tbl = t0_hbm.at[c] if l == 0 else out_hbm.at[l - 1].at[c]

        def issue_gather(k, b):
            return pltpu.async_copy(tbl.at[src_v.at[k]], rows[b], sem_g[b])

        def scale(k, b):
            @pl.loop(0, E_CHUNK)
            def _(r):
                widx = jnp.full((16,), k * E_CHUNK + r, dtype=jnp.int32)
                wspl = plsc.load_gather(w_v, [widx])
                rows[b][r, pl.ds(0, 16)] = rows[b][r, pl.ds(0, 16)] * wspl
                rows[b][r, pl.ds(16, 16)] = rows[b][r, pl.ds(16, 16)] * wspl

        def issue_scatter(k, b):
            return pltpu.async_copy(rows[b], acc.at[dst_v.at[k]], sem_s[b],
                                    add=True)

        @pl.loop(0, E_SB_PER_TILE)
        def _(sb):
            sbc = s * E_SB_PER_TILE + sb
            pltpu.sync_copy(src_hbm.at[pl.ds(sbc * SB_CHUNKS, SB_CHUNKS)], src_v)
            pltpu.sync_copy(dst_hbm.at[pl.ds(sbc * SB_CHUNKS, SB_CHUNKS)], dst_v)
            pltpu.sync_copy(w_hbm.at[pl.ds(sbc * SB_EDGES, SB_EDGES)], w_v)

            gd = [issue_gather(b, b) for b in range(NBUF)]
            sc = [None] * NBUF
            for k in range(SB_CHUNKS):
                b = k % NBUF
                gd[b].wait()
                scale(k, b)
                sc[b] = issue_scatter(k, b)
                if k + NBUF < SB_CHUNKS:
                    sc[b].wait()
                    gd[b] = issue_gather(k + NBUF, b)
            for b in range(NBUF):
                sc[b].wait()

        plsc.subcore_barrier()

        @pl.loop(0, ROWS_PER_TILE // ZCHUNK)
        def _(z):
            r0 = s * ROWS_PER_TILE + z * ZCHUNK
            pltpu.sync_copy(acc.at[pl.ds(r0, ZCHUNK)],
                            out_hbm.at[l].at[c].at[pl.ds(r0, ZCHUNK)])
            pltpu.sync_copy(z_hbm, acc.at[pl.ds(r0, ZCHUNK)])

        plsc.subcore_barrier()


_BN = 1000


def _mean_body(t0_ref, outs_ref, o_ref):
    t0 = t0_ref[...]
    outs = outs_ref[...]
    m = (t0 + outs[0] + outs[1] + outs[2]) * 0.25
    o_ref[...] = jnp.concatenate([m[0], m[1]], axis=-1)


def _tc_mean(t0, outs):
    grid = (NUM_NODES // _BN,)
    return pl.pallas_call(
        _mean_body,
        grid=grid,
        in_specs=[
            pl.BlockSpec((NC, _BN, HD), lambda i: (0, i, 0)),
            pl.BlockSpec((N_LAYERS, NC, _BN, HD), lambda i: (0, 0, i, 0)),
        ],
        out_specs=pl.BlockSpec((_BN, D), lambda i: (i, 0)),
        out_shape=jax.ShapeDtypeStruct((NUM_NODES, D), jnp.float32),
    )(t0, outs)


def kernel(user_emb, item_emb, entity_emb, relation_emb, fc_W, fc_b,
           edge_weight, edge_index, kg_tails, kg_relations):
    tails_pad = jnp.pad(kg_tails, ((0, I_PAD - NUM_ITEMS), (0, 0)))
    idx_flat = tails_pad.reshape(G_PAD // 128, 128)
    ent_g = _sc_gather_entities(entity_emb, idx_flat)
    ent3r = ent_g.reshape(I_PAD, K, D)

    fc_b2 = fc_b.reshape(1, D)
    rel_pad = jnp.pad(relation_emb, ((0, 24 - relation_emb.shape[0]), (0, 0)))
    item_kg = _kg_attention(ent3r, item_emb, kg_relations, fc_W, fc_b2, rel_pad)

    all_emb = jnp.concatenate([user_emb, item_kg], axis=0)
    t0 = jnp.stack([all_emb[:, :HD], all_emb[:, HD:]])

    pad_e = E_PAD - edge_weight.shape[0]
    src = jnp.pad(edge_index[0], (0, pad_e)).reshape(E_PAD // 128, 128)
    dst = jnp.pad(edge_index[1], (0, pad_e)).reshape(E_PAD // 128, 128)
    w = jnp.pad(edge_weight, (0, pad_e))
    z = jnp.zeros((ZCHUNK, HD), jnp.float32)

    outs = _sc_propagate(t0, src, dst, w, z)

    light = _tc_mean(t0, outs)
    return light[:NUM_USERS], light[NUM_USERS:]


# speedup vs baseline: 1.0673x; 1.0673x over previous
import functools

import jax
import jax.numpy as jnp
from jax import lax
from jax.experimental import pallas as pl
from jax.experimental.pallas import tpu as pltpu
from jax.experimental.pallas import tpu_sc as plsc

NUM_USERS = 35000
NUM_ITEMS = 15000
NUM_NODES = NUM_USERS + NUM_ITEMS
D = 64
HD = 32
K = 10
N_LAYERS = 3

NC = 2
NS = 16
NW = NC * NS

I_PAD = 16384
G_PAD = I_PAD * K
G_CHUNK = 512
G_SUB = G_CHUNK // 128
G_CHUNKS_PER_W = G_PAD // (NW * G_CHUNK)

E_PAD = 802816
E_CHUNK = 128
NBUF = 4
SB_CHUNKS = 28
E_SB_PER_TILE = 14
SB_EDGES = SB_CHUNKS * E_CHUNK
ROWS_PER_TILE = NUM_NODES // NS
ZCHUNK = 625

_mesh = plsc.VectorSubcoreMesh(
    core_axis_name="c", subcore_axis_name="s", num_cores=NC, num_subcores=NS
)
_sc_params = pltpu.CompilerParams(
    use_tc_tiling_on_sc=False, needs_layout_passes=False
)


@functools.partial(
    pl.kernel,
    mesh=_mesh,
    out_type=jax.ShapeDtypeStruct((G_PAD, D), jnp.float32),
    scratch_types=[
        pltpu.VMEM((G_CHUNKS_PER_W * G_SUB, 128), jnp.int32),
        [pltpu.VMEM((G_CHUNK, D), jnp.float32) for _ in range(2)],
        [pltpu.SemaphoreType.DMA for _ in range(2)],
        [pltpu.SemaphoreType.DMA for _ in range(2)],
    ],
    compiler_params=_sc_params,
)
def _sc_gather_entities(tbl_hbm, idx_hbm, out_hbm, idx_v, rows, sem_g, sem_w):
    c = lax.axis_index("c")
    s = lax.axis_index("s")
    wid = s * NC + c

    pltpu.sync_copy(
        idx_hbm.at[pl.ds(wid * G_CHUNKS_PER_W * G_SUB, G_CHUNKS_PER_W * G_SUB)],
        idx_v)

    def fire_gathers(j, b):
        return [pltpu.async_copy(tbl_hbm.at[idx_v.at[j * G_SUB + jj]],
                                 rows[b].at[pl.ds(jj * 128, 128)],
                                 sem_g[b])
                for jj in range(G_SUB)]

    gd = [fire_gathers(0, 0), fire_gathers(1, 1)]
    wb = [None, None]
    for j in range(G_CHUNKS_PER_W):
        b = j % 2
        for d_ in gd[b]:
            d_.wait()
        base = (wid * G_CHUNKS_PER_W + j) * G_CHUNK
        wb[b] = pltpu.async_copy(rows[b], out_hbm.at[pl.ds(base, G_CHUNK)],
                                 sem_w[b])
        if j + 2 < G_CHUNKS_PER_W:
            wb[b].wait()
            gd[b] = fire_gathers(j + 2, b)
    wb[0].wait()
    wb[1].wait()


_BI = 512
I_GRID = 30
I_PAD2 = I_GRID * _BI


def _kg_body(*refs):
    ent_refs = refs[:K]
    item_ref, rel_ref, w_ref, b_ref, re_ref, o_ref = refs[K:]
    hi = jax.lax.Precision.HIGHEST
    item = item_ref[...]
    W1 = w_ref[0:D, :]
    W2 = w_ref[D : 2 * D, :]
    b = b_ref[...]
    re_tab = re_ref[...]
    itemW1 = lax.dot(item, W1, precision=hi)
    kgr = rel_ref[...]

    ents = [r[...] for r in ent_refs]
    es = []
    for k in range(K):
        entW2 = lax.dot(ents[k], W2, precision=hi)
        iota24 = lax.broadcasted_iota(jnp.int32, (_BI, 24), 1)
        oh = (kgr[:, k : k + 1] == iota24).astype(jnp.float32)
        rel_k = lax.dot(oh, re_tab, precision=hi)
        e_k = jnp.sum((itemW1 + entW2 + b) * rel_k, axis=-1, keepdims=True)
        es.append(e_k)
    e = jnp.concatenate(es, axis=1)
    e = jnp.where(e > 0, e, 0.2 * e)
    m = jnp.max(e, axis=1, keepdims=True)
    p = jnp.exp(e - m)
    att = p / jnp.sum(p, axis=1, keepdims=True)

    out = item
    for k in range(K):
        out = out + att[:, k : k + 1] * ents[k]
    o_ref[0] = out[:, :HD]
    o_ref[1] = out[:, HD:]


def _kg_attention(ent_g, item_emb, kg_relations, fc_W, fc_b2, rel_pad):
    ent_specs = [
        pl.BlockSpec((_BI, D), lambda i, k=k: (k * (I_PAD // _BI) + i, 0))
        for k in range(K)
    ]
    return pl.pallas_call(
        _kg_body,
        grid=(I_GRID,),
        in_specs=ent_specs + [
            pl.BlockSpec((_BI, D), lambda i: (i, 0)),
            pl.BlockSpec((_BI, K), lambda i: (i, 0)),
            pl.BlockSpec((2 * D, D), lambda i: (0, 0)),
            pl.BlockSpec((1, D), lambda i: (0, 0)),
            pl.BlockSpec((24, D), lambda i: (0, 0)),
        ],
        out_specs=pl.BlockSpec((NC, _BI, HD), lambda i: (0, i, 0)),
        out_shape=jax.ShapeDtypeStruct((NC, I_PAD2, HD), jnp.float32),
    )(*([ent_g] * K), item_emb, kg_relations, fc_W, fc_b2, rel_pad)


@functools.partial(
    pl.kernel,
    mesh=_mesh,
    out_type=jax.ShapeDtypeStruct((N_LAYERS, NC, NUM_NODES, HD), jnp.float32),
    scratch_types=[
        pltpu.VMEM((SB_CHUNKS, 128), jnp.int32),
        pltpu.VMEM((SB_CHUNKS, 128), jnp.int32),
        pltpu.VMEM((SB_EDGES,), jnp.float32),
        [pltpu.VMEM((E_CHUNK, HD), jnp.float32) for _ in range(NBUF)],
        pltpu.VMEM_SHARED((NUM_NODES, HD), jnp.float32),
        [pltpu.SemaphoreType.DMA for _ in range(NBUF)],
        [pltpu.SemaphoreType.DMA for _ in range(NBUF)],
    ],
    compiler_params=_sc_params,
)
def _sc_propagate(t0_hbm, src_hbm, dst_hbm, w_hbm, z_hbm, out_hbm,
                  src_v, dst_v, w_v, rows, acc, sem_g, sem_s):
    c = lax.axis_index("c")
    s = lax.axis_index("s")

    @pl.loop(0, ROWS_PER_TILE // ZCHUNK)
    def _(z):
        pltpu.sync_copy(z_hbm, acc.at[pl.ds(s * ROWS_PER_TILE + z * ZCHUNK, ZCHUNK)])

    plsc.subcore_barrier()

    for l in range(N_LAYERS):
        tbl = t0_hbm.at[c] if l == 0 else out_hbm.at[l - 1].at[c]

        def issue_gather(k, b):
            return pltpu.async_copy(tbl.at[src_v.at[k]], rows[b], sem_g[b])

        def scale(k, b):
            @pl.loop(0, E_CHUNK)
            def _(r):
                widx = jnp.full((16,), k * E_CHUNK + r, dtype=jnp.int32)
                wspl = plsc.load_gather(w_v, [widx])
                rows[b][r, pl.ds(0, 16)] = rows[b][r, pl.ds(0, 16)] * wspl
                rows[b][r, pl.ds(16, 16)] = rows[b][r, pl.ds(16, 16)] * wspl

        def issue_scatter(k, b):
            return pltpu.async_copy(rows[b], acc.at[dst_v.at[k]], sem_s[b],
                                    add=True)

        @pl.loop(0, E_SB_PER_TILE)
        def _(sb):
            sbc = s * E_SB_PER_TILE + sb
            pltpu.sync_copy(src_hbm.at[pl.ds(sbc * SB_CHUNKS, SB_CHUNKS)], src_v)
            pltpu.sync_copy(dst_hbm.at[pl.ds(sbc * SB_CHUNKS, SB_CHUNKS)], dst_v)
            pltpu.sync_copy(w_hbm.at[pl.ds(sbc * SB_EDGES, SB_EDGES)], w_v)

            gd = [issue_gather(b, b) for b in range(NBUF)]
            sc = [None] * NBUF
            for k in range(SB_CHUNKS):
                b = k % NBUF
                gd[b].wait()
                scale(k, b)
                sc[b] = issue_scatter(k, b)
                if k + NBUF < SB_CHUNKS:
                    sc[b].wait()
                    gd[b] = issue_gather(k + NBUF, b)
            for b in range(NBUF):
                sc[b].wait()

        plsc.subcore_barrier()

        @pl.loop(0, ROWS_PER_TILE // ZCHUNK)
        def _(z):
            r0 = s * ROWS_PER_TILE + z * ZCHUNK
            pltpu.sync_copy(acc.at[pl.ds(r0, ZCHUNK)],
                            out_hbm.at[l].at[c].at[pl.ds(r0, ZCHUNK)])
            pltpu.sync_copy(z_hbm, acc.at[pl.ds(r0, ZCHUNK)])

        plsc.subcore_barrier()


_BN = 1000


def _mean_body(t0_ref, outs_ref, o_ref):
    t0 = t0_ref[...]
    outs = outs_ref[...]
    m = (t0 + outs[0] + outs[1] + outs[2]) * 0.25
    o_ref[...] = jnp.concatenate([m[0], m[1]], axis=-1)


def _tc_mean(t0, outs):
    grid = (NUM_NODES // _BN,)
    return pl.pallas_call(
        _mean_body,
        grid=grid,
        in_specs=[
            pl.BlockSpec((NC, _BN, HD), lambda i: (0, i, 0)),
            pl.BlockSpec((N_LAYERS, NC, _BN, HD), lambda i: (0, 0, i, 0)),
        ],
        out_specs=pl.BlockSpec((_BN, D), lambda i: (i, 0)),
        out_shape=jax.ShapeDtypeStruct((NUM_NODES, D), jnp.float32),
    )(t0, outs)


def kernel(user_emb, item_emb, entity_emb, relation_emb, fc_W, fc_b,
           edge_weight, edge_index, kg_tails, kg_relations):
    tails_t = jnp.pad(kg_tails.T, ((0, 0), (0, I_PAD - NUM_ITEMS)))
    idx_flat = tails_t.reshape(G_PAD // 128, 128)
    ent_g = _sc_gather_entities(entity_emb, idx_flat)

    fc_b2 = fc_b.reshape(1, D)
    rel_pad = jnp.pad(relation_emb, ((0, 24 - relation_emb.shape[0]), (0, 0)))
    item_h = _kg_attention(ent_g, item_emb, kg_relations, fc_W, fc_b2, rel_pad)

    user_h = jnp.stack([user_emb[:, :HD], user_emb[:, HD:]])
    t0 = jnp.concatenate([user_h, item_h[:, :NUM_ITEMS, :]], axis=1)

    pad_e = E_PAD - edge_weight.shape[0]
    src = jnp.pad(edge_index[0], (0, pad_e)).reshape(E_PAD // 128, 128)
    dst = jnp.pad(edge_index[1], (0, pad_e)).reshape(E_PAD // 128, 128)
    w = jnp.pad(edge_weight, (0, pad_e))
    z = jnp.zeros((ZCHUNK, HD), jnp.float32)

    outs = _sc_propagate(t0, src, dst, w, z)

    light = _tc_mean(t0, outs)
    return light[:NUM_USERS], light[NUM_USERS:]
